# baseline (speedup 1.0000x reference)
import jax
import jax.numpy as jnp
from jax.experimental import pallas as pl
from jax.experimental.pallas import tpu as pltpu

_D_MODEL = 1024
_D_EMBED = 128
_N_CHR = 32
_ECC_BITS = 32
_K = 32
_SEQ = 2048
_R = 256
_NB = _SEQ // _R


def _fused_kernel(x_ref, we_ref, be_ref, wd_ref, wc_ref, bc_ref,
                  wp_ref, bp_ref, w1_ref, b1_ref, w2_ref, b2_ref,
                  emb_ref, met_ref, chr_ref, ecc_ref,
                  scores_ref, idx_ref, minh_ref, maxh_ref,
                  embn_ref):
    p = pl.program_id(0)
    b = pl.program_id(1)
    s = pl.program_id(2)
    row0 = (b * _NB + s) * _R

    @pl.when(p == 0)
    def _phase0():
        x = x_ref[0]
        emb = jnp.dot(x, we_ref[...], preferred_element_type=jnp.float32) + be_ref[...]
        nrm = jnp.sqrt(jnp.sum(emb * emb, axis=1, keepdims=True)) + 1e-8
        embn_ref[pl.ds(row0, _R), :] = emb / nrm

    @pl.when(p == 1)
    def _phase1():
        x = x_ref[0]
        emb = jnp.dot(x, we_ref[...], preferred_element_type=jnp.float32) + be_ref[...]
        emb_ref[0] = emb
        met_ref[0] = jnp.dot(x, wd_ref[...], preferred_element_type=jnp.float32)
        chr_ref[0] = jnp.dot(x, wc_ref[...], preferred_element_type=jnp.float32) + bc_ref[...]
        pr = jnp.dot(x, wp_ref[...], preferred_element_type=jnp.float32) + bp_ref[...]
        h = jnp.tanh(jnp.dot(pr, w1_ref[...], preferred_element_type=jnp.float32) + b1_ref[...])
        ecc_ref[0] = jax.nn.sigmoid(
            jnp.dot(h, w2_ref[...], preferred_element_type=jnp.float32) + b2_ref[...])

        q = embn_ref[pl.ds(row0, _R), :]
        km = embn_ref[pl.ds(b * _SEQ, _SEQ), :]
        sim = jax.lax.dot_general(q, km, (((1,), (1,)), ((), ())),
                                  preferred_element_type=jnp.float32)
        rows = jax.lax.broadcasted_iota(jnp.int32, (_R, _SEQ), 0) + s * _R
        cols = jax.lax.broadcasted_iota(jnp.int32, (_R, _SEQ), 1)
        work = jnp.where(rows == cols, jnp.float32(-1e9), sim)

        s_list, i_list = [], []
        for _ in range(_K):
            m = jnp.max(work, axis=1, keepdims=True)
            loc = work == m
            amin = jnp.min(jnp.where(loc, cols, _SEQ), axis=1, keepdims=True)
            s_list.append(m)
            i_list.append(amin)
            work = jnp.where(loc, jnp.float32(-2e9), work)
        scores = jnp.concatenate(s_list, axis=1)
        idx = jnp.concatenate(i_list, axis=1)
        scores_ref[0] = scores
        idx_ref[0] = idx
        minh_ref[0] = scores[:, :_K // 2]
        maxh_ref[0] = -scores[:, _K // 2:]


def kernel(x, W_embed, b_embed, W_diag, W_chr, b_chr,
           W_ecc_proj, b_ecc_proj, W_e1, b_e1, W_e2, b_e2):
    B, S, D = x.shape
    f32 = jnp.float32

    be = b_embed.reshape(1, -1)
    bc = b_chr.reshape(1, -1)
    bp = b_ecc_proj.reshape(1, -1)
    b1 = b_e1.reshape(1, -1)
    b2 = b_e2.reshape(1, -1)

    full = lambda shp: pl.BlockSpec(shp, lambda p, b, s: (0,) * len(shp))
    blk = lambda w: pl.BlockSpec((1, _R, w), lambda p, b, s: (b, s, 0))
    emb, met, chrs, ecc, scores, idx, minh, maxh = pl.pallas_call(
        _fused_kernel,
        grid=(2, B, _NB),
        in_specs=[
            pl.BlockSpec((1, _R, D), lambda p, b, s: (b, s, 0)),
            full((D, _D_EMBED)), full((1, _D_EMBED)),
            full((D, D)),
            full((D, _N_CHR)), full((1, _N_CHR)),
            full((D, _ECC_BITS)), full((1, _ECC_BITS)),
            full((_ECC_BITS, 2 * _ECC_BITS)), full((1, 2 * _ECC_BITS)),
            full((2 * _ECC_BITS, _ECC_BITS)), full((1, _ECC_BITS)),
        ],
        out_specs=[
            blk(_D_EMBED), blk(D), blk(_N_CHR), blk(_ECC_BITS),
            blk(_K), blk(_K), blk(_K // 2), blk(_K // 2),
        ],
        out_shape=[
            jax.ShapeDtypeStruct((B, S, _D_EMBED), f32),
            jax.ShapeDtypeStruct((B, S, D), f32),
            jax.ShapeDtypeStruct((B, S, _N_CHR), f32),
            jax.ShapeDtypeStruct((B, S, _ECC_BITS), f32),
            jax.ShapeDtypeStruct((B, S, _K), f32),
            jax.ShapeDtypeStruct((B, S, _K), jnp.int32),
            jax.ShapeDtypeStruct((B, S, _K // 2), f32),
            jax.ShapeDtypeStruct((B, S, _K // 2), f32),
        ],
        scratch_shapes=[pltpu.VMEM((B * S, _D_EMBED), f32)],
        compiler_params=pltpu.CompilerParams(
            dimension_semantics=("arbitrary", "arbitrary", "arbitrary")),
    )(x, W_embed, be, W_diag, W_chr, bc, W_ecc_proj, bp, W_e1, b1, W_e2, b2)

    return (emb, met, chrs, scores, idx, minh, maxh, ecc)


# speedup vs baseline: 1.1704x; 1.1704x over previous
import jax
import jax.numpy as jnp
from jax.experimental import pallas as pl
from jax.experimental.pallas import tpu as pltpu

_D_MODEL = 1024
_D_EMBED = 128
_N_CHR = 32
_ECC_BITS = 32
_K = 32
_SEQ = 2048
_R = 256
_NB = _SEQ // _R


def _fused_kernel(x_ref, we_ref, be_ref, wd_ref, wc_ref, bc_ref,
                  wp_ref, bp_ref, w1_ref, b1_ref, w2_ref, b2_ref,
                  emb_ref, met_ref, chr_ref, ecc_ref,
                  scores_ref, idx_ref, minh_ref, maxh_ref,
                  embn_ref):
    p = pl.program_id(0)
    b = pl.program_id(1)
    s = pl.program_id(2)
    row0 = (b * _NB + s) * _R

    @pl.when(p == 0)
    def _phase0():
        x = x_ref[0]
        emb = jnp.dot(x, we_ref[...], preferred_element_type=jnp.float32) + be_ref[...]
        nrm = jnp.sqrt(jnp.sum(emb * emb, axis=1, keepdims=True)) + 1e-8
        embn_ref[pl.ds(row0, _R), :] = emb / nrm

    @pl.when(p == 1)
    def _phase1():
        x = x_ref[0]
        emb = jnp.dot(x, we_ref[...], preferred_element_type=jnp.float32) + be_ref[...]
        emb_ref[0] = emb
        met_ref[0] = jnp.dot(x, wd_ref[...], preferred_element_type=jnp.float32)
        chr_ref[0] = jnp.dot(x, wc_ref[...], preferred_element_type=jnp.float32) + bc_ref[...]
        pr = jnp.dot(x, wp_ref[...], preferred_element_type=jnp.float32) + bp_ref[...]
        h = jnp.tanh(jnp.dot(pr, w1_ref[...], preferred_element_type=jnp.float32) + b1_ref[...])
        ecc_ref[0] = jax.nn.sigmoid(
            jnp.dot(h, w2_ref[...], preferred_element_type=jnp.float32) + b2_ref[...])

        q = embn_ref[pl.ds(row0, _R), :]
        km = embn_ref[pl.ds(b * _SEQ, _SEQ), :]
        sim = jax.lax.dot_general(q, km, (((1,), (1,)), ((), ())),
                                  preferred_element_type=jnp.float32)
        rows = jax.lax.broadcasted_iota(jnp.int32, (_R, _SEQ), 0) + s * _R
        cols = jax.lax.broadcasted_iota(jnp.int32, (_R, _SEQ), 1)
        colsf = cols.astype(jnp.float32)
        work = jnp.where(rows == cols, jnp.float32(-1e9), sim)

        s_list, i_list = [], []
        for _ in range(_K):
            m = jnp.max(work, axis=1, keepdims=True)
            loc = work == m
            amin = jnp.min(jnp.where(loc, colsf, jnp.float32(_SEQ)),
                           axis=1, keepdims=True)
            s_list.append(m)
            i_list.append(amin)
            work = jnp.where(loc, jnp.float32(-2e9), work)
        scores = jnp.concatenate(s_list, axis=1)
        idx = jnp.concatenate(i_list, axis=1).astype(jnp.int32)
        scores_ref[0] = scores
        idx_ref[0] = idx
        minh_ref[0] = scores[:, :_K // 2]
        maxh_ref[0] = -scores[:, _K // 2:]


def kernel(x, W_embed, b_embed, W_diag, W_chr, b_chr,
           W_ecc_proj, b_ecc_proj, W_e1, b_e1, W_e2, b_e2):
    B, S, D = x.shape
    f32 = jnp.float32

    be = b_embed.reshape(1, -1)
    bc = b_chr.reshape(1, -1)
    bp = b_ecc_proj.reshape(1, -1)
    b1 = b_e1.reshape(1, -1)
    b2 = b_e2.reshape(1, -1)

    full = lambda shp: pl.BlockSpec(shp, lambda p, b, s: (0,) * len(shp))
    blk = lambda w: pl.BlockSpec((1, _R, w), lambda p, b, s: (b, s, 0))
    emb, met, chrs, ecc, scores, idx, minh, maxh = pl.pallas_call(
        _fused_kernel,
        grid=(2, B, _NB),
        in_specs=[
            pl.BlockSpec((1, _R, D), lambda p, b, s: (b, s, 0)),
            full((D, _D_EMBED)), full((1, _D_EMBED)),
            full((D, D)),
            full((D, _N_CHR)), full((1, _N_CHR)),
            full((D, _ECC_BITS)), full((1, _ECC_BITS)),
            full((_ECC_BITS, 2 * _ECC_BITS)), full((1, 2 * _ECC_BITS)),
            full((2 * _ECC_BITS, _ECC_BITS)), full((1, _ECC_BITS)),
        ],
        out_specs=[
            blk(_D_EMBED), blk(D), blk(_N_CHR), blk(_ECC_BITS),
            blk(_K), blk(_K), blk(_K // 2), blk(_K // 2),
        ],
        out_shape=[
            jax.ShapeDtypeStruct((B, S, _D_EMBED), f32),
            jax.ShapeDtypeStruct((B, S, D), f32),
            jax.ShapeDtypeStruct((B, S, _N_CHR), f32),
            jax.ShapeDtypeStruct((B, S, _ECC_BITS), f32),
            jax.ShapeDtypeStruct((B, S, _K), f32),
            jax.ShapeDtypeStruct((B, S, _K), jnp.int32),
            jax.ShapeDtypeStruct((B, S, _K // 2), f32),
            jax.ShapeDtypeStruct((B, S, _K // 2), f32),
        ],
        scratch_shapes=[pltpu.VMEM((B * S, _D_EMBED), f32)],
        compiler_params=pltpu.CompilerParams(
            dimension_semantics=("arbitrary", "arbitrary", "arbitrary")),
    )(x, W_embed, be, W_diag, W_chr, bc, W_ecc_proj, bp, W_e1, b1, W_e2, b2)

    return (emb, met, chrs, scores, idx, minh, maxh, ecc)
